# baseline (speedup 1.0000x reference)
import functools

import jax
import jax.numpy as jnp
from jax import lax
from jax.experimental import pallas as pl
from jax.experimental.pallas import tpu as pltpu
from jax.experimental.pallas import tpu_sc as plsc

_TILE = 4096


def _gather_sc(emb, x):
  B = x.shape[0]
  E = emb.shape[1]
  info = plsc.get_sparse_core_info()
  nw = info.num_cores * info.num_subcores
  b_per_w = B // nw
  mesh = plsc.VectorSubcoreMesh(core_axis_name="c", subcore_axis_name="s")

  @functools.partial(
      pl.kernel,
      mesh=mesh,
      out_type=jax.ShapeDtypeStruct((B, E), jnp.float32),
      scratch_types=[
          pltpu.VMEM((b_per_w,), jnp.int32),
          pltpu.VMEM((b_per_w, E), jnp.float32),
          pltpu.SemaphoreType.DMA,
      ],
  )
  def gather(table_hbm, idx_hbm, out_hbm, idx_v, rows_v, sem):
    wid = lax.axis_index("s") * info.num_cores + lax.axis_index("c")
    base = wid * b_per_w
    pltpu.sync_copy(idx_hbm.at[pl.ds(base, b_per_w)], idx_v)
    pltpu.async_copy(table_hbm.at[idx_v], rows_v, sem).wait()
    pltpu.sync_copy(rows_v, out_hbm.at[pl.ds(base, b_per_w)])

  return gather(emb, x)


def _stats_body(nv):
  def body(e_ref, w_ref, b_ref, lse_ref):
    j = pl.program_id(0)
    t = lax.dot_general(
        e_ref[...],
        w_ref[...],
        (((1,), (1,)), ((), ())),
        preferred_element_type=jnp.float32,
    ) + b_ref[...]

    @pl.when(j == 0)
    def _():
      lse_ref[...] = jnp.zeros(lse_ref.shape, lse_ref.dtype)

    lse_ref[...] += jnp.sum(jnp.exp(t), axis=1, keepdims=True)

    @pl.when(j == nv - 1)
    def _():
      lse_ref[...] = jnp.log(lse_ref[...])

  return body


def _out_body(e_ref, w_ref, b_ref, lse_ref, out_ref):
  t = lax.dot_general(
      e_ref[...],
      w_ref[...],
      (((1,), (1,)), ((), ())),
      preferred_element_type=jnp.float32,
  )
  out_ref[...] = t + (b_ref[...] - lse_ref[...])


def kernel(x, emb, W, b):
  V, E = W.shape
  B = x.shape[0]
  nv = pl.cdiv(V, _TILE)
  Vp = nv * _TILE
  e = _gather_sc(emb, x.astype(jnp.int32)).astype(jnp.bfloat16)
  Wp = jnp.pad(W.astype(jnp.bfloat16), ((0, Vp - V), (0, 0)))
  bp = jnp.pad(b.reshape(1, V), ((0, 0), (0, Vp - V)), constant_values=-1e9)

  lse = pl.pallas_call(
      _stats_body(nv),
      grid=(nv,),
      in_specs=[
          pl.BlockSpec((B, E), lambda j: (0, 0)),
          pl.BlockSpec((_TILE, E), lambda j: (j, 0)),
          pl.BlockSpec((1, _TILE), lambda j: (0, j)),
      ],
      out_specs=pl.BlockSpec((B, 1), lambda j: (0, 0)),
      out_shape=jax.ShapeDtypeStruct((B, 1), jnp.float32),
  )(e, Wp, bp)

  out = pl.pallas_call(
      _out_body,
      grid=(nv,),
      in_specs=[
          pl.BlockSpec((B, E), lambda j: (0, 0)),
          pl.BlockSpec((_TILE, E), lambda j: (j, 0)),
          pl.BlockSpec((1, _TILE), lambda j: (0, j)),
          pl.BlockSpec((B, 1), lambda j: (0, 0)),
      ],
      out_specs=pl.BlockSpec((B, _TILE), lambda j: (0, j)),
      out_shape=jax.ShapeDtypeStruct((B, V), jnp.float32),
      compiler_params=pltpu.CompilerParams(
          dimension_semantics=("parallel",)
      ),
  )(e, Wp, bp, lse)
  return out


# speedup vs baseline: 2.1036x; 2.1036x over previous
import functools

import jax
import jax.numpy as jnp
from jax import lax
from jax.experimental import pallas as pl
from jax.experimental.pallas import tpu as pltpu
from jax.experimental.pallas import tpu_sc as plsc

_TILE = 4096


def _gather_sc(emb, x):
  B = x.shape[0]
  E = emb.shape[1]
  info = plsc.get_sparse_core_info()
  nw = info.num_cores * info.num_subcores
  b_per_w = B // nw
  mesh = plsc.VectorSubcoreMesh(core_axis_name="c", subcore_axis_name="s")

  @functools.partial(
      pl.kernel,
      mesh=mesh,
      out_type=jax.ShapeDtypeStruct((B, E), jnp.float32),
      scratch_types=[
          pltpu.VMEM((b_per_w,), jnp.int32),
          pltpu.VMEM((b_per_w, E), jnp.float32),
          pltpu.SemaphoreType.DMA,
      ],
  )
  def gather(table_hbm, idx_hbm, out_hbm, idx_v, rows_v, sem):
    wid = lax.axis_index("s") * info.num_cores + lax.axis_index("c")
    base = wid * b_per_w
    pltpu.sync_copy(idx_hbm.at[pl.ds(base, b_per_w)], idx_v)
    pltpu.async_copy(table_hbm.at[idx_v], rows_v, sem).wait()
    pltpu.sync_copy(rows_v, out_hbm.at[pl.ds(base, b_per_w)])

  return gather(emb, x)


def _stats_body(nv):
  def body(e_ref, w_ref, b_ref, lse_ref):
    j = pl.program_id(0)
    t = lax.dot_general(
        e_ref[...],
        w_ref[...],
        (((1,), (1,)), ((), ())),
        preferred_element_type=jnp.float32,
    ) + b_ref[...]

    @pl.when(j == 0)
    def _():
      lse_ref[...] = jnp.zeros(lse_ref.shape, lse_ref.dtype)

    lse_ref[...] += jnp.sum(jnp.exp(t), axis=1, keepdims=True)

    @pl.when(j == nv - 1)
    def _():
      lse_ref[...] = jnp.log(lse_ref[...])

  return body


def _out_body(e_ref, w_ref, b_ref, lse_ref, out_ref):
  t = lax.dot_general(
      e_ref[...],
      w_ref[...],
      (((1,), (1,)), ((), ())),
      preferred_element_type=jnp.float32,
  )
  out_ref[0] = t + (b_ref[...] - lse_ref[...])


def kernel(x, emb, W, b):
  V, E = W.shape
  B = x.shape[0]
  nv = pl.cdiv(V, _TILE)
  Vp = nv * _TILE
  e = _gather_sc(emb, x.astype(jnp.int32)).astype(jnp.bfloat16)
  Wp = jnp.pad(W.astype(jnp.bfloat16), ((0, Vp - V), (0, 0)))
  bp = jnp.pad(b.reshape(1, V), ((0, 0), (0, Vp - V)), constant_values=-1e9)

  lse = pl.pallas_call(
      _stats_body(nv),
      grid=(nv,),
      in_specs=[
          pl.BlockSpec((B, E), lambda j: (0, 0)),
          pl.BlockSpec((_TILE, E), lambda j: (j, 0)),
          pl.BlockSpec((1, _TILE), lambda j: (0, j)),
      ],
      out_specs=pl.BlockSpec((B, 1), lambda j: (0, 0)),
      out_shape=jax.ShapeDtypeStruct((B, 1), jnp.float32),
  )(e, Wp, bp)

  out = pl.pallas_call(
      _out_body,
      grid=(nv,),
      in_specs=[
          pl.BlockSpec((B, E), lambda j: (0, 0)),
          pl.BlockSpec((_TILE, E), lambda j: (j, 0)),
          pl.BlockSpec((1, _TILE), lambda j: (0, j)),
          pl.BlockSpec((B, 1), lambda j: (0, 0)),
      ],
      out_specs=pl.BlockSpec((1, B, _TILE), lambda j: (j, 0, 0)),
      out_shape=jax.ShapeDtypeStruct((nv, B, _TILE), jnp.float32),
      compiler_params=pltpu.CompilerParams(
          dimension_semantics=("parallel",)
      ),
  )(e, Wp, bp, lse)
  return out
